# baseline (speedup 1.0000x reference)
import functools
import math

import jax
import jax.numpy as jnp
from jax import lax
from jax.experimental import pallas as pl
from jax.experimental.pallas import tpu as pltpu
from jax.experimental.pallas import tpu_sc as plsc

D_MODEL = 64
VOCAB = 1000000
OUT_W = 128
SCALE = math.sqrt(D_MODEL)

_NC, _NS, _LANES = 2, 16, 16
_NW = _NC * _NS

_MESH = plsc.VectorSubcoreMesh(core_axis_name="c", subcore_axis_name="s")

_K1_ROWS = 128
_SLAB = VOCAB // 2
_K1_SLAB_BLOCKS = -(-_SLAB // _K1_ROWS)
_K1_BLOCKS = 2 * _K1_SLAB_BLOCKS
_K1_LAST_OFF = _SLAB - _K1_ROWS
_K1_STEPS = -(-_K1_BLOCKS // _NW)
_K1_FLAT = _K1_ROWS * D_MODEL


@functools.partial(
    pl.kernel,
    mesh=_MESH,
    out_type=jax.ShapeDtypeStruct((VOCAB * D_MODEL,), jnp.float32),
    scratch_types=[
        pltpu.VMEM((2, _K1_ROWS, D_MODEL), jnp.float32),
        pltpu.VMEM((3 * _K1_FLAT,), jnp.float32),
        pltpu.SemaphoreType.DMA,
        pltpu.SemaphoreType.DMA,
    ],
)
def _k1_depad_scale(tab_hbm, out_hbm, inb, flat, rsem, wsem):
    wid = lax.axis_index("s") * _NC + lax.axis_index("c")

    def blk_parts(step):
        blk = step * _NW + wid
        slab = blk // _K1_SLAB_BLOCKS
        off = jnp.minimum((blk % _K1_SLAB_BLOCKS) * _K1_ROWS, _K1_LAST_OFF)
        return slab, off

    def valid(step):
        return (step * _NW + wid) < _K1_BLOCKS

    def start_read(step, b):
        slab, off = blk_parts(step)
        pltpu.async_copy(
            tab_hbm.at[slab, pl.ds(off, _K1_ROWS), :], inb.at[b], rsem
        )

    def wait_read(b):
        pltpu.make_async_copy(
            tab_hbm.at[0, pl.ds(0, _K1_ROWS), :], inb.at[b], rsem
        ).wait()

    def start_write(step, b):
        slab, off = blk_parts(step)
        pltpu.async_copy(
            flat.at[pl.ds(b * _K1_FLAT, _K1_FLAT)],
            out_hbm.at[pl.ds((slab * _SLAB + off) * D_MODEL, _K1_FLAT)],
            wsem,
        )

    def wait_write(b):
        pltpu.make_async_copy(
            flat.at[pl.ds(b * _K1_FLAT, _K1_FLAT)], out_hbm.at[pl.ds(0, _K1_FLAT)], wsem
        ).wait()

    def compact_scale(rb, fb):
        for r in range(_K1_ROWS):
            for j in range(D_MODEL // _LANES):
                vals = inb[rb, r, pl.ds(j * _LANES, _LANES)]
                flat[pl.ds(fb * _K1_FLAT + r * D_MODEL + j * _LANES, _LANES)] = vals * SCALE

    @pl.when(valid(0))
    def _():
        start_read(0, 0)

    def step_body(i, carry):
        for k in range(6):
            s = i * 6 + k
            rb, fb = k % 2, k % 3

            @pl.when(valid(s))
            def _():
                @pl.when(valid(s + 1))
                def _():
                    start_read(s + 1, (k + 1) % 2)

                wait_read(rb)

                @pl.when(s >= 3)
                def _():
                    wait_write(fb)

                compact_scale(rb, fb)
                start_write(s, fb)

        return carry

    lax.fori_loop(0, (_K1_STEPS + 5) // 6, step_body, 0)
    wait_write(0)
    wait_write(1)
    wait_write(2)


_CHUNK = 512
_NBUF = 2


def _make_k2_gather(B: int):
    assert B % (_NW * _CHUNK * _NBUF) == 0
    b_per_w = B // _NW
    chunks = b_per_w // _CHUNK
    groups = chunks // _NBUF

    @functools.partial(
        pl.kernel,
        mesh=_MESH,
        out_type=jax.ShapeDtypeStruct((B, OUT_W), jnp.float32),
        scratch_types=[
            pltpu.VMEM((_NBUF, _CHUNK), jnp.int32),
            pltpu.VMEM((_NBUF, _CHUNK, D_MODEL), jnp.float32),
            pltpu.SemaphoreType.DMA,
            pltpu.SemaphoreType.DMA,
        ],
        compiler_params=pltpu.CompilerParams(use_tc_tiling_on_sc=False),
    )
    def k2_gather(x_hbm, table_hbm, out_hbm, idx_v, rows_v, gsem, osem):
        wid = lax.axis_index("s") * _NC + lax.axis_index("c")
        base = wid * b_per_w

        def start_gather(c, b):
            off = base + c * _CHUNK
            pltpu.sync_copy(x_hbm.at[pl.ds(off, _CHUNK)], idx_v.at[b])
            pltpu.async_copy(table_hbm.at[idx_v.at[b]], rows_v.at[b], gsem)

        def wait_gather(b):
            pltpu.make_async_copy(
                table_hbm.at[idx_v.at[b]], rows_v.at[b], gsem
            ).wait()

        def start_out(c, b):
            off = base + c * _CHUNK
            pltpu.async_copy(
                rows_v.at[b],
                out_hbm.at[pl.ds(off, _CHUNK), pl.ds(0, D_MODEL)],
                osem,
            )

        def wait_out(b):
            pltpu.make_async_copy(
                rows_v.at[b],
                out_hbm.at[pl.ds(0, _CHUNK), pl.ds(0, D_MODEL)],
                osem,
            ).wait()

        start_gather(0, 0)

        def group_body(g, carry):
            for b in range(_NBUF):
                c = g * _NBUF + b
                o = (b + 1) % _NBUF

                @pl.when(c + 1 < chunks)
                def _():
                    @pl.when(c >= 1)
                    def _():
                        wait_out(o)

                    start_gather(c + 1, o)

                wait_gather(b)
                start_out(c, b)
            return carry

        lax.fori_loop(0, groups, group_body, 0)
        for b in range(min(_NBUF, chunks)):
            wait_out(b)

    return k2_gather


_K2_GATHER = _make_k2_gather(4096 * 200)


def kernel(x, table):
    orig_shape = x.shape
    x_flat = x.reshape((-1,)).astype(jnp.int32)
    table_scaled = _k1_depad_scale(table.reshape(2, _SLAB, D_MODEL))
    out = _K2_GATHER(x_flat, table_scaled.reshape(VOCAB, D_MODEL))
    out = out.reshape(orig_shape + (OUT_W,))
    return out[..., :D_MODEL]


# speedup vs baseline: 1.0819x; 1.0819x over previous
import functools
import math

import jax
import jax.numpy as jnp
from jax import lax
from jax.experimental import pallas as pl
from jax.experimental.pallas import tpu as pltpu
from jax.experimental.pallas import tpu_sc as plsc

D_MODEL = 64
VOCAB = 1000000
OUT_W = 128
SCALE = math.sqrt(D_MODEL)

_NC, _NS, _LANES = 2, 16, 16
_NW = _NC * _NS

_MESH = plsc.VectorSubcoreMesh(core_axis_name="c", subcore_axis_name="s")

_K1_ROWS = 128
_SLAB = VOCAB // 2
_K1_SLAB_BLOCKS = -(-_SLAB // _K1_ROWS)
_K1_BLOCKS = 2 * _K1_SLAB_BLOCKS
_K1_LAST_OFF = _SLAB - _K1_ROWS
_K1_STEPS = -(-_K1_BLOCKS // _NW)
_K1_FLAT = _K1_ROWS * D_MODEL


@functools.partial(
    pl.kernel,
    mesh=_MESH,
    out_type=jax.ShapeDtypeStruct((VOCAB * D_MODEL,), jnp.float32),
    scratch_types=[
        pltpu.VMEM((2, _K1_ROWS, D_MODEL), jnp.float32),
        pltpu.VMEM((2 * _K1_FLAT,), jnp.float32),
        pltpu.SemaphoreType.DMA,
        pltpu.SemaphoreType.DMA,
    ],
)
def _k1_depad_scale(tab_hbm, out_hbm, inb, flat, rsem, wsem):
    wid = lax.axis_index("s") * _NC + lax.axis_index("c")

    def blk_parts(step):
        blk = step * _NW + wid
        slab = blk // _K1_SLAB_BLOCKS
        off = jnp.minimum((blk % _K1_SLAB_BLOCKS) * _K1_ROWS, _K1_LAST_OFF)
        return slab, off

    def valid(step):
        return (step * _NW + wid) < _K1_BLOCKS

    def start_read(step, b):
        slab, off = blk_parts(step)
        pltpu.async_copy(
            tab_hbm.at[slab, pl.ds(off, _K1_ROWS), :], inb.at[b], rsem
        )

    def wait_read(b):
        pltpu.make_async_copy(
            tab_hbm.at[0, pl.ds(0, _K1_ROWS), :], inb.at[b], rsem
        ).wait()

    def start_write(step, b):
        slab, off = blk_parts(step)
        pltpu.async_copy(
            flat.at[pl.ds(b * _K1_FLAT, _K1_FLAT)],
            out_hbm.at[pl.ds((slab * _SLAB + off) * D_MODEL, _K1_FLAT)],
            wsem,
        )

    def wait_write(b):
        pltpu.make_async_copy(
            flat.at[pl.ds(b * _K1_FLAT, _K1_FLAT)], out_hbm.at[pl.ds(0, _K1_FLAT)], wsem
        ).wait()

    def compact_scale(rb, fb):
        for r in range(_K1_ROWS):
            for j in range(D_MODEL // _LANES):
                vals = inb[rb, r, pl.ds(j * _LANES, _LANES)]
                flat[pl.ds(fb * _K1_FLAT + r * D_MODEL + j * _LANES, _LANES)] = vals * SCALE

    @pl.when(valid(0))
    def _():
        start_read(0, 0)

    def step_body(i, carry):
        for k in range(2):
            s = i * 2 + k
            rb = fb = k

            @pl.when(valid(s))
            def _():
                @pl.when(valid(s + 1))
                def _():
                    start_read(s + 1, (k + 1) % 2)

                wait_read(rb)

                @pl.when(s >= 2)
                def _():
                    wait_write(fb)

                compact_scale(rb, fb)
                start_write(s, fb)

        return carry

    lax.fori_loop(0, (_K1_STEPS + 1) // 2, step_body, 0)
    wait_write(0)
    wait_write(1)


_CHUNK = 512
_NBUF = 2


def _make_k2_gather(B: int):
    assert B % (_NW * _CHUNK * _NBUF) == 0
    b_per_w = B // _NW
    chunks = b_per_w // _CHUNK
    groups = chunks // _NBUF

    @functools.partial(
        pl.kernel,
        mesh=_MESH,
        out_type=jax.ShapeDtypeStruct((B, OUT_W), jnp.float32),
        scratch_types=[
            pltpu.VMEM((_NBUF, _CHUNK), jnp.int32),
            pltpu.VMEM((_NBUF, _CHUNK, D_MODEL), jnp.float32),
            pltpu.SemaphoreType.DMA,
            pltpu.SemaphoreType.DMA,
        ],
        compiler_params=pltpu.CompilerParams(use_tc_tiling_on_sc=False),
    )
    def k2_gather(x_hbm, table_hbm, out_hbm, idx_v, rows_v, gsem, osem):
        wid = lax.axis_index("s") * _NC + lax.axis_index("c")
        base = wid * b_per_w

        def start_gather(c, b):
            off = base + c * _CHUNK
            pltpu.sync_copy(x_hbm.at[pl.ds(off, _CHUNK)], idx_v.at[b])
            pltpu.async_copy(table_hbm.at[idx_v.at[b]], rows_v.at[b], gsem)

        def wait_gather(b):
            pltpu.make_async_copy(
                table_hbm.at[idx_v.at[b]], rows_v.at[b], gsem
            ).wait()

        def start_out(c, b):
            off = base + c * _CHUNK
            pltpu.async_copy(
                rows_v.at[b],
                out_hbm.at[pl.ds(off, _CHUNK), pl.ds(0, D_MODEL)],
                osem,
            )

        def wait_out(b):
            pltpu.make_async_copy(
                rows_v.at[b],
                out_hbm.at[pl.ds(0, _CHUNK), pl.ds(0, D_MODEL)],
                osem,
            ).wait()

        start_gather(0, 0)

        def group_body(g, carry):
            for b in range(_NBUF):
                c = g * _NBUF + b
                o = (b + 1) % _NBUF

                @pl.when(c + 1 < chunks)
                def _():
                    @pl.when(c >= 1)
                    def _():
                        wait_out(o)

                    start_gather(c + 1, o)

                wait_gather(b)
                start_out(c, b)
            return carry

        lax.fori_loop(0, groups, group_body, 0)
        for b in range(min(_NBUF, chunks)):
            wait_out(b)

    return k2_gather


_K2_GATHER = _make_k2_gather(4096 * 200)


def kernel(x, table):
    orig_shape = x.shape
    x_flat = x.reshape((-1,)).astype(jnp.int32)
    table_scaled = _k1_depad_scale(table.reshape(2, _SLAB, D_MODEL))
    out = _K2_GATHER(x_flat, table_scaled.reshape(VOCAB, D_MODEL))
    out = out.reshape(orig_shape + (OUT_W,))
    return out[..., :D_MODEL]


# speedup vs baseline: 1.0896x; 1.0072x over previous
import functools
import math

import jax
import jax.numpy as jnp
from jax import lax
from jax.experimental import pallas as pl
from jax.experimental.pallas import tpu as pltpu
from jax.experimental.pallas import tpu_sc as plsc

D_MODEL = 64
VOCAB = 1000000
OUT_W = 128
SCALE = math.sqrt(D_MODEL)

_NC, _NS, _LANES = 2, 16, 16
_NW = _NC * _NS

_MESH = plsc.VectorSubcoreMesh(core_axis_name="c", subcore_axis_name="s")

_K1_ROWS = 128
_SLAB = VOCAB // 2
_K1_SLAB_BLOCKS = -(-_SLAB // _K1_ROWS)
_K1_BLOCKS = 2 * _K1_SLAB_BLOCKS
_K1_LAST_OFF = _SLAB - _K1_ROWS
_K1_STEPS = -(-_K1_BLOCKS // _NW)
_K1_FLAT = _K1_ROWS * D_MODEL


@functools.partial(
    pl.kernel,
    mesh=_MESH,
    out_type=jax.ShapeDtypeStruct((VOCAB * D_MODEL,), jnp.float32),
    scratch_types=[
        pltpu.VMEM((2, _K1_ROWS, D_MODEL), jnp.float32),
        pltpu.VMEM((2 * _K1_FLAT,), jnp.float32),
        pltpu.SemaphoreType.DMA,
        pltpu.SemaphoreType.DMA,
    ],
)
def _k1_depad_scale(tab_hbm, out_hbm, inb, flat, rsem, wsem):
    wid = lax.axis_index("s") * _NC + lax.axis_index("c")

    def blk_parts(step):
        blk = step * _NW + wid
        slab = blk // _K1_SLAB_BLOCKS
        off = jnp.minimum((blk % _K1_SLAB_BLOCKS) * _K1_ROWS, _K1_LAST_OFF)
        return slab, off

    def valid(step):
        return (step * _NW + wid) < _K1_BLOCKS

    def start_read(step, b):
        slab, off = blk_parts(step)
        pltpu.async_copy(
            tab_hbm.at[slab, pl.ds(off, _K1_ROWS), :], inb.at[b], rsem
        )

    def wait_read(b):
        pltpu.make_async_copy(
            tab_hbm.at[0, pl.ds(0, _K1_ROWS), :], inb.at[b], rsem
        ).wait()

    def start_write(step, b):
        slab, off = blk_parts(step)
        pltpu.async_copy(
            flat.at[pl.ds(b * _K1_FLAT, _K1_FLAT)],
            out_hbm.at[pl.ds((slab * _SLAB + off) * D_MODEL, _K1_FLAT)],
            wsem,
        )

    def wait_write(b):
        pltpu.make_async_copy(
            flat.at[pl.ds(b * _K1_FLAT, _K1_FLAT)], out_hbm.at[pl.ds(0, _K1_FLAT)], wsem
        ).wait()

    def compact_scale(rb, fb):
        for r in range(_K1_ROWS):
            for j in range(D_MODEL // _LANES):
                vals = inb[rb, r, pl.ds(j * _LANES, _LANES)]
                flat[pl.ds(fb * _K1_FLAT + r * D_MODEL + j * _LANES, _LANES)] = vals * SCALE

    @pl.when(valid(0))
    def _():
        start_read(0, 0)

    def step_body(i, carry):
        for k in range(2):
            s = i * 2 + k
            rb = fb = k

            @pl.when(valid(s))
            def _():
                @pl.when(valid(s + 1))
                def _():
                    start_read(s + 1, (k + 1) % 2)

                wait_read(rb)

                @pl.when(s >= 2)
                def _():
                    wait_write(fb)

                compact_scale(rb, fb)
                start_write(s, fb)

        return carry

    lax.fori_loop(0, (_K1_STEPS + 1) // 2, step_body, 0)
    wait_write(0)
    wait_write(1)


_CHUNK = 800
_NBUF = 2


def _make_k2_gather(B: int):
    assert B % (_NW * _CHUNK * _NBUF) == 0, B
    b_per_w = B // _NW
    chunks = b_per_w // _CHUNK
    groups = chunks // _NBUF

    @functools.partial(
        pl.kernel,
        mesh=_MESH,
        out_type=jax.ShapeDtypeStruct((B, OUT_W), jnp.float32),
        scratch_types=[
            pltpu.VMEM((_NBUF, _CHUNK), jnp.int32),
            pltpu.VMEM((_NBUF, _CHUNK, D_MODEL), jnp.float32),
            pltpu.SemaphoreType.DMA,
            pltpu.SemaphoreType.DMA,
        ],
        compiler_params=pltpu.CompilerParams(use_tc_tiling_on_sc=False),
    )
    def k2_gather(x_hbm, table_hbm, out_hbm, idx_v, rows_v, gsem, osem):
        wid = lax.axis_index("s") * _NC + lax.axis_index("c")
        base = wid * b_per_w

        def start_gather(c, b):
            off = base + c * _CHUNK
            pltpu.sync_copy(x_hbm.at[pl.ds(off, _CHUNK)], idx_v.at[b])
            pltpu.async_copy(table_hbm.at[idx_v.at[b]], rows_v.at[b], gsem)

        def wait_gather(b):
            pltpu.make_async_copy(
                table_hbm.at[idx_v.at[b]], rows_v.at[b], gsem
            ).wait()

        def start_out(c, b):
            off = base + c * _CHUNK
            pltpu.async_copy(
                rows_v.at[b],
                out_hbm.at[pl.ds(off, _CHUNK), pl.ds(0, D_MODEL)],
                osem,
            )

        def wait_out(b):
            pltpu.make_async_copy(
                rows_v.at[b],
                out_hbm.at[pl.ds(0, _CHUNK), pl.ds(0, D_MODEL)],
                osem,
            ).wait()

        start_gather(0, 0)

        def group_body(g, carry):
            for b in range(_NBUF):
                c = g * _NBUF + b
                o = (b + 1) % _NBUF

                @pl.when(c + 1 < chunks)
                def _():
                    @pl.when(c >= 1)
                    def _():
                        wait_out(o)

                    start_gather(c + 1, o)

                wait_gather(b)
                start_out(c, b)
            return carry

        lax.fori_loop(0, groups, group_body, 0)
        for b in range(min(_NBUF, chunks)):
            wait_out(b)

    return k2_gather


_K2_GATHER = _make_k2_gather(4096 * 200)


def kernel(x, table):
    orig_shape = x.shape
    x_flat = x.reshape((-1,)).astype(jnp.int32)
    table_scaled = _k1_depad_scale(table.reshape(2, _SLAB, D_MODEL))
    out = _K2_GATHER(x_flat, table_scaled.reshape(VOCAB, D_MODEL))
    out = out.reshape(orig_shape + (OUT_W,))
    return out[..., :D_MODEL]
